# baseline (speedup 1.0000x reference)
import functools

import jax
import jax.numpy as jnp
from jax import lax
from jax.experimental import pallas as pl
from jax.experimental.pallas import tpu as pltpu
from jax.experimental.pallas import tpu_sc as plsc

BETA = 0.25
M_TILE = 576
N_CHUNK = 4096


def _vq_argmin_body(z_ref, cb_ref, cn_ref, idx_ref, loss_ref, m_ref, zz_ref):
    n_cb = cb_ref.shape[0]
    n_k = n_cb // 128
    zt = z_ref[...]
    zz = jnp.sum(zt * zt, axis=1, keepdims=True)
    zneg = zt * (-2.0)
    m_ref[...] = lax.dot_general(zneg, cb_ref[...], (((1,), (1,)), ((), ())),
                                 preferred_element_type=jnp.float32)
    zz_ref[...] = zz
    lane = lax.broadcasted_iota(jnp.int32, (8, 128), 1)

    def row_group(r, acc):
        zzr = zz_ref[pl.ds(r * 8, 8), :]
        run_min = jnp.full((8, 128), jnp.inf, jnp.float32)
        run_k = jnp.zeros((8, 128), jnp.int32)
        for k in range(n_k):
            sk = (zzr + m_ref[pl.ds(r * 8, 8), k * 128:(k + 1) * 128]) \
                + cn_ref[k:k + 1, :]
            upd = sk < run_min
            run_min = jnp.where(upd, sk, run_min)
            run_k = jnp.where(upd, jnp.int32(k), run_k)
        gmin = jnp.min(run_min, axis=1, keepdims=True)
        cand = run_k * 128 + lane
        bidx = jnp.min(jnp.where(run_min == gmin, cand, jnp.int32(2**30)),
                       axis=1, keepdims=True)
        idx_ref[pl.ds(r * 8, 8), :] = bidx
        return acc + gmin

    acc = lax.fori_loop(0, M_TILE // 8, row_group,
                        jnp.zeros((8, 1), jnp.float32))

    part = jnp.sum(acc, keepdims=True)
    prev = jnp.where(pl.program_id(0) == 0,
                     jnp.zeros((1, 1), jnp.float32), loss_ref[...])
    total_rows = pl.num_programs(0) * M_TILE
    scale = (1.0 + BETA) / total_rows
    loss_ref[...] = prev + part * scale


def _vq_argmin(zf, codebook):
    n_rows = zf.shape[0]
    grid = (n_rows // M_TILE,)
    return pl.pallas_call(
        _vq_argmin_body,
        grid=grid,
        in_specs=[
            pl.BlockSpec((M_TILE, zf.shape[1]), lambda i: (i, 0)),
            pl.BlockSpec(codebook.shape, lambda i: (0, 0)),
            pl.BlockSpec((codebook.shape[0] // 128, 128), lambda i: (0, 0)),
        ],
        out_specs=[
            pl.BlockSpec((M_TILE, 1), lambda i: (i, 0)),
            pl.BlockSpec((1, 1), lambda i: (0, 0)),
        ],
        out_shape=[
            jax.ShapeDtypeStruct((n_rows, 1), jnp.int32),
            jax.ShapeDtypeStruct((1, 1), jnp.float32),
        ],
        scratch_shapes=[
            pltpu.VMEM((M_TILE, codebook.shape[0]), jnp.float32),
            pltpu.VMEM((M_TILE, 1), jnp.float32),
        ],
    )(zf, codebook, jnp.sum(codebook * codebook, axis=1).reshape(-1, 128))


def _sc_gather(table, idx_flat, d_out_cols):
    n_rows = idx_flat.shape[0]
    dt = table.shape[1]
    d_out = d_out_cols
    info = plsc.get_sparse_core_info()
    nw = info.num_cores * info.num_subcores
    b_per_w = n_rows // nw
    n_chunks = 2
    chunk = b_per_w // n_chunks
    mesh = plsc.VectorSubcoreMesh(core_axis_name="c", subcore_axis_name="s")

    @functools.partial(
        pl.kernel,
        mesh=mesh,
        out_type=jax.ShapeDtypeStruct((n_rows, dt), jnp.float32),
        scratch_types=[
            pltpu.VMEM((n_chunks, chunk), jnp.int32),
            pltpu.VMEM((b_per_w, dt), jnp.float32),
            pltpu.SemaphoreType.DMA,
        ],
    )
    def gather_kernel(table_hbm, idx_hbm, out_hbm, idx_v, rows_v, sem):
        wid = lax.axis_index("s") * info.num_cores + lax.axis_index("c")
        base = wid * b_per_w
        for j in range(n_chunks):
            pltpu.sync_copy(idx_hbm.at[pl.ds(base + j * chunk, chunk)],
                            idx_v.at[j])
        copies = [
            pltpu.async_copy(table_hbm.at[idx_v.at[j]],
                             rows_v.at[pl.ds(j * chunk, chunk)], sem)
            for j in range(n_chunks)
        ]
        for c in copies:
            c.wait()
        pltpu.sync_copy(rows_v, out_hbm.at[pl.ds(base, b_per_w)])

    return gather_kernel(table, idx_flat)


def kernel(z, codebook):
    b, t, d = z.shape
    zf = z.reshape(-1, d)
    idx2d, loss = _vq_argmin(zf, codebook)
    idx_flat = idx2d.reshape(-1)
    table = jnp.concatenate([codebook, jnp.zeros_like(codebook)], axis=1)
    q = _sc_gather(table, idx_flat, d)[:, :d]
    return q.reshape(b, t, d), loss[0, 0], idx_flat.reshape(b, t)


# speedup vs baseline: 3.6740x; 3.6740x over previous
import functools

import jax
import jax.numpy as jnp
from jax import lax
from jax.experimental import pallas as pl
from jax.experimental.pallas import tpu as pltpu
from jax.experimental.pallas import tpu_sc as plsc

BETA = 0.25
M_TILE = 576
N_CHUNK = 4096


def _vq_argmin_body(z_ref, cb_ref, cn_ref, idx_ref, loss_ref, m_ref, zz_ref):
    n_cb = cb_ref.shape[0]
    n_k = n_cb // 128
    R = 96
    zt = z_ref[...]
    zz = jnp.sum(zt * zt, axis=1, keepdims=True)
    zneg = zt * (-2.0)
    m_ref[...] = lax.dot_general(zneg, cb_ref[...], (((1,), (1,)), ((), ())),
                                 preferred_element_type=jnp.float32)
    zz_ref[...] = zz + jnp.zeros((M_TILE, 128), jnp.float32)
    lane = lax.broadcasted_iota(jnp.int32, (R, 128), 1)

    acc = jnp.zeros((1, 1), jnp.float32)
    for r in range(M_TILE // R):
        zzr = zz_ref[r * R:(r + 1) * R, :]
        run_min = jnp.full((R, 128), jnp.inf, jnp.float32)
        run_k = jnp.zeros((R, 128), jnp.int32)
        for k in range(n_k):
            sk = (zzr + m_ref[r * R:(r + 1) * R, k * 128:(k + 1) * 128]) \
                + cn_ref[k:k + 1, :]
            upd = sk < run_min
            run_min = jnp.where(upd, sk, run_min)
            run_k = jnp.where(upd, jnp.int32(k), run_k)
        gmin = jnp.min(run_min, axis=1, keepdims=True)
        cand = run_k * 128 + lane
        bidx = jnp.min(jnp.where(run_min == gmin, cand, jnp.int32(2**30)),
                       axis=1, keepdims=True)
        idx_ref[r * R:(r + 1) * R, :] = bidx
        acc = acc + jnp.sum(gmin, keepdims=True)

    part = acc
    prev = jnp.where(pl.program_id(0) == 0,
                     jnp.zeros((1, 1), jnp.float32), loss_ref[...])
    total_rows = pl.num_programs(0) * M_TILE
    scale = (1.0 + BETA) / total_rows
    loss_ref[...] = prev + part * scale


def _vq_argmin(zf, codebook):
    n_rows = zf.shape[0]
    grid = (n_rows // M_TILE,)
    return pl.pallas_call(
        _vq_argmin_body,
        grid=grid,
        in_specs=[
            pl.BlockSpec((M_TILE, zf.shape[1]), lambda i: (i, 0)),
            pl.BlockSpec(codebook.shape, lambda i: (0, 0)),
            pl.BlockSpec((codebook.shape[0] // 128, 128), lambda i: (0, 0)),
        ],
        out_specs=[
            pl.BlockSpec((M_TILE, 1), lambda i: (i, 0)),
            pl.BlockSpec((1, 1), lambda i: (0, 0)),
        ],
        out_shape=[
            jax.ShapeDtypeStruct((n_rows, 1), jnp.int32),
            jax.ShapeDtypeStruct((1, 1), jnp.float32),
        ],
        scratch_shapes=[
            pltpu.VMEM((M_TILE, codebook.shape[0]), jnp.float32),
            pltpu.VMEM((M_TILE, 128), jnp.float32),
        ],
    )(zf, codebook, jnp.sum(codebook * codebook, axis=1).reshape(-1, 128))


def _sc_gather(table, idx_flat, d_out_cols):
    n_rows = idx_flat.shape[0]
    dt = table.shape[1]
    d_out = d_out_cols
    info = plsc.get_sparse_core_info()
    nw = info.num_cores * info.num_subcores
    b_per_w = n_rows // nw
    n_chunks = 2
    chunk = b_per_w // n_chunks
    mesh = plsc.VectorSubcoreMesh(core_axis_name="c", subcore_axis_name="s")

    @functools.partial(
        pl.kernel,
        mesh=mesh,
        out_type=jax.ShapeDtypeStruct((n_rows, dt), jnp.float32),
        scratch_types=[
            pltpu.VMEM((n_chunks, chunk), jnp.int32),
            pltpu.VMEM((b_per_w, dt), jnp.float32),
            pltpu.SemaphoreType.DMA,
        ],
    )
    def gather_kernel(table_hbm, idx_hbm, out_hbm, idx_v, rows_v, sem):
        wid = lax.axis_index("s") * info.num_cores + lax.axis_index("c")
        base = wid * b_per_w
        for j in range(n_chunks):
            pltpu.sync_copy(idx_hbm.at[pl.ds(base + j * chunk, chunk)],
                            idx_v.at[j])
        copies = [
            pltpu.async_copy(table_hbm.at[idx_v.at[j]],
                             rows_v.at[pl.ds(j * chunk, chunk)], sem)
            for j in range(n_chunks)
        ]
        for c in copies:
            c.wait()
        pltpu.sync_copy(rows_v, out_hbm.at[pl.ds(base, b_per_w)])

    return gather_kernel(table, idx_flat)


def kernel(z, codebook):
    b, t, d = z.shape
    zf = z.reshape(-1, d)
    idx2d, loss = _vq_argmin(zf, codebook)
    idx_flat = idx2d.reshape(-1)
    table = jnp.concatenate([codebook, jnp.zeros_like(codebook)], axis=1)
    q = _sc_gather(table, idx_flat, d)[:, :d]
    return q.reshape(b, t, d), loss[0, 0], idx_flat.reshape(b, t)


# speedup vs baseline: 3.6898x; 1.0043x over previous
import functools

import jax
import jax.numpy as jnp
from jax import lax
from jax.experimental import pallas as pl
from jax.experimental.pallas import tpu as pltpu
from jax.experimental.pallas import tpu_sc as plsc

BETA = 0.25
M_TILE = 576
N_CHUNK = 4096


def _vq_argmin_body(z_ref, cb_ref, cn_ref, idx_ref, loss_ref, tab_ref,
                    m_ref, zz_ref):
    n_cb = cb_ref.shape[0]
    n_k = n_cb // 128
    R = 96

    @pl.when(pl.program_id(0) == 0)
    def _():
        tab_ref[:, 0:64] = cb_ref[...]

    zt = z_ref[...]
    zz = jnp.sum(zt * zt, axis=1, keepdims=True)
    zneg = zt * (-2.0)
    m_ref[...] = lax.dot_general(zneg, cb_ref[...], (((1,), (1,)), ((), ())),
                                 preferred_element_type=jnp.float32)
    zz_ref[...] = zz + jnp.zeros((M_TILE, 128), jnp.float32)
    lane = lax.broadcasted_iota(jnp.int32, (R, 128), 1)

    acc = jnp.zeros((1, 1), jnp.float32)
    for r in range(M_TILE // R):
        zzr = zz_ref[r * R:(r + 1) * R, :]
        run_min = jnp.full((R, 128), jnp.inf, jnp.float32)
        run_k = jnp.zeros((R, 128), jnp.int32)
        for k in range(n_k):
            sk = (zzr + m_ref[r * R:(r + 1) * R, k * 128:(k + 1) * 128]) \
                + cn_ref[k:k + 1, :]
            upd = sk < run_min
            run_min = jnp.where(upd, sk, run_min)
            run_k = jnp.where(upd, jnp.int32(k), run_k)
        gmin = jnp.min(run_min, axis=1, keepdims=True)
        cand = run_k * 128 + lane
        bidx = jnp.min(jnp.where(run_min == gmin, cand, jnp.int32(2**30)),
                       axis=1, keepdims=True)
        idx_ref[r * R:(r + 1) * R, :] = bidx
        acc = acc + jnp.sum(gmin, keepdims=True)

    part = acc
    prev = jnp.where(pl.program_id(0) == 0,
                     jnp.zeros((1, 1), jnp.float32), loss_ref[...])
    total_rows = pl.num_programs(0) * M_TILE
    scale = (1.0 + BETA) / total_rows
    loss_ref[...] = prev + part * scale


def _vq_argmin(zf, codebook):
    n_rows = zf.shape[0]
    grid = (n_rows // M_TILE,)
    return pl.pallas_call(
        _vq_argmin_body,
        grid=grid,
        in_specs=[
            pl.BlockSpec((M_TILE, zf.shape[1]), lambda i: (i, 0)),
            pl.BlockSpec(codebook.shape, lambda i: (0, 0)),
            pl.BlockSpec((codebook.shape[0] // 128, 128), lambda i: (0, 0)),
        ],
        out_specs=[
            pl.BlockSpec((M_TILE, 1), lambda i: (i, 0)),
            pl.BlockSpec((1, 1), lambda i: (0, 0)),
            pl.BlockSpec((codebook.shape[0], 128), lambda i: (0, 0)),
        ],
        out_shape=[
            jax.ShapeDtypeStruct((n_rows, 1), jnp.int32),
            jax.ShapeDtypeStruct((1, 1), jnp.float32),
            jax.ShapeDtypeStruct((codebook.shape[0], 128), jnp.float32),
        ],
        scratch_shapes=[
            pltpu.VMEM((M_TILE, codebook.shape[0]), jnp.float32),
            pltpu.VMEM((M_TILE, 128), jnp.float32),
        ],
    )(zf, codebook, jnp.sum(codebook * codebook, axis=1).reshape(-1, 128))


def _sc_gather(table, idx_flat, d_out_cols):
    n_rows = idx_flat.shape[0]
    dt = table.shape[1]
    d_out = d_out_cols
    info = plsc.get_sparse_core_info()
    nw = info.num_cores * info.num_subcores
    b_per_w = n_rows // nw
    n_chunks = 2
    chunk = b_per_w // n_chunks
    mesh = plsc.VectorSubcoreMesh(core_axis_name="c", subcore_axis_name="s")

    @functools.partial(
        pl.kernel,
        mesh=mesh,
        out_type=jax.ShapeDtypeStruct((n_rows, dt), jnp.float32),
        scratch_types=[
            pltpu.VMEM((n_chunks, chunk), jnp.int32),
            pltpu.VMEM((b_per_w, dt), jnp.float32),
            pltpu.SemaphoreType.DMA,
        ],
    )
    def gather_kernel(table_hbm, idx_hbm, out_hbm, idx_v, rows_v, sem):
        wid = lax.axis_index("s") * info.num_cores + lax.axis_index("c")
        base = wid * b_per_w
        for j in range(n_chunks):
            pltpu.sync_copy(idx_hbm.at[pl.ds(base + j * chunk, chunk)],
                            idx_v.at[j])
        copies = [
            pltpu.async_copy(table_hbm.at[idx_v.at[j]],
                             rows_v.at[pl.ds(j * chunk, chunk)], sem)
            for j in range(n_chunks)
        ]
        for c in copies:
            c.wait()
        pltpu.sync_copy(rows_v, out_hbm.at[pl.ds(base, b_per_w)])

    return gather_kernel(table, idx_flat)


def kernel(z, codebook):
    b, t, d = z.shape
    zf = z.reshape(-1, d)
    idx2d, loss, table = _vq_argmin(zf, codebook)
    idx_flat = idx2d.reshape(-1)
    q = _sc_gather(table, idx_flat, d)[:, :d]
    return q.reshape(b, t, d), loss[0, 0], idx_flat.reshape(b, t)
